# baseline (speedup 1.0000x reference)
import functools

import jax
import jax.numpy as jnp
from jax import lax
from jax.experimental import pallas as pl
from jax.experimental.pallas import tpu as pltpu
from jax.experimental.pallas import tpu_sc as plsc

N = 10000
E = 160000
P = 20000
IN, HID, OUT = 128, 256, 128

NC, NS, L = 2, 16, 16
NW = NC * NS
K = 128
NBUF = 2

R = 10240
ECH0, ECH1 = 56, 24
TCH = (ECH0 + ECH1) * NS
EPAD = TCH * K
SCH0, SCH1 = 10, 0
SPAD = (SCH0 + SCH1) * NS * K
RPT = R // NS
DW = 1


def _seg_body(s0, d0, s1, d1, t0, t1, zer, zer8, one8,
              agg_out, deg_out,
              agg_sh, deg_sh, sidx, didx, r0, r1, o1v, gsem,
              ssem0, ssem1, dsem):
    c = lax.axis_index("c")
    s = lax.axis_index("s")
    rows = (r0, r1)
    ssems = (ssem0, ssem1)
    rbase = s * RPT
    cbase = jnp.where(c == 0, s * ECH0, NS * ECH0 + s * ECH1)
    nch = jnp.where(c == 0, ECH0, ECH1)
    ngrp = nch // NBUF

    pltpu.sync_copy(one8, o1v)
    pltpu.sync_copy(zer, agg_sh.at[pl.ds(rbase, RPT)])
    pltpu.sync_copy(zer8, deg_sh.at[pl.ds(rbase, RPT)])
    plsc.subcore_barrier()

    for rel in range(2):
        src = (s0, s1)[rel]
        dst = (d0, d1)[rel]
        tab = (t0, t1)[rel]

        @pl.when(c == 0)
        def _():
            pltpu.sync_copy(src.at[pl.ds(cbase, ECH0)], sidx)
            pltpu.sync_copy(dst.at[pl.ds(cbase, ECH0)], didx)

        @pl.when(c == 1)
        def _():
            pltpu.sync_copy(src.at[pl.ds(cbase, ECH1)], sidx.at[pl.ds(0, ECH1)])
            pltpu.sync_copy(dst.at[pl.ds(cbase, ECH1)], didx.at[pl.ds(0, ECH1)])

        pltpu.async_copy(tab.at[sidx.at[0]], r0, gsem)

        def chunk_group(p, carry, tab=tab):
            for b in range(NBUF):
                j = NBUF * p + b
                ob = 1 - b
                pltpu.make_async_copy(tab.at[sidx.at[j]], rows[b],
                                      gsem).wait()

                @pl.when(j >= 1)
                def _(ob=ob, j=j):
                    pltpu.make_async_copy(rows[ob],
                                          agg_sh.at[didx.at[j - 1]],
                                          ssems[ob]).wait()
                    pltpu.make_async_copy(o1v, deg_sh.at[didx.at[j - 1]],
                                          dsem).wait()

                pltpu.async_copy(rows[b], agg_sh.at[didx.at[j]], ssems[b],
                                 add=True)
                pltpu.async_copy(o1v, deg_sh.at[didx.at[j]], dsem, add=True)

                @pl.when(j + 1 < nch)
                def _(ob=ob, j=j, tab=tab):
                    pltpu.async_copy(tab.at[sidx.at[j + 1]], rows[ob], gsem)
            return carry

        lax.fori_loop(0, ngrp, chunk_group, 0)
        pltpu.make_async_copy(r1, agg_sh.at[didx.at[0]], ssem1).wait()
        pltpu.make_async_copy(o1v, deg_sh.at[didx.at[0]], dsem).wait()
        plsc.subcore_barrier()
        pltpu.sync_copy(agg_sh.at[pl.ds(rbase, RPT)],
                        agg_out.at[rel, c, pl.ds(rbase, RPT)])
        pltpu.sync_copy(deg_sh.at[pl.ds(rbase, RPT)],
                        deg_out.at[rel, c, pl.ds(rbase, RPT)])
        if rel == 0:
            pltpu.sync_copy(zer, agg_sh.at[pl.ds(rbase, RPT)])
            pltpu.sync_copy(zer8, deg_sh.at[pl.ds(rbase, RPT)])
            plsc.subcore_barrier()


_seg_call = functools.partial(
    pl.kernel,
    _seg_body,
    out_type=(
        jax.ShapeDtypeStruct((2, NC, R, IN), jnp.float32),
        jax.ShapeDtypeStruct((2, NC, R), jnp.float32),
    ),
    mesh=plsc.VectorSubcoreMesh(core_axis_name="c", subcore_axis_name="s",
                                num_cores=NC, num_subcores=NS),
    scratch_types=[
        pltpu.VMEM_SHARED((R, IN), jnp.float32),
        pltpu.VMEM_SHARED((R,), jnp.float32),
        pltpu.VMEM((ECH0, K), jnp.int32),
        pltpu.VMEM((ECH0, K), jnp.int32),
        pltpu.VMEM((K, IN), jnp.float32),
        pltpu.VMEM((K, IN), jnp.float32),
        pltpu.VMEM((K,), jnp.float32),
        pltpu.SemaphoreType.DMA,
        pltpu.SemaphoreType.DMA,
        pltpu.SemaphoreType.DMA,
        pltpu.SemaphoreType.DMA,
    ],
)()


def _score_body(z, u0, v0, u1, v1, u2, v2, u3, v3,
                ou0, ov0, ou1, ov1, ou2, ov2, ou3, ov3,
                uidx, vidx, zu, zv, gsem):
    c = lax.axis_index("c")
    s = lax.axis_index("s")
    cbase = jnp.where(c == 0, s * SCH0, NS * SCH0 + s * SCH1)
    nch = jnp.where(c == 0, SCH0, SCH1)

    for u, v, ou, ov in ((u0, v0, ou0, ov0), (u1, v1, ou1, ov1),
                         (u2, v2, ou2, ov2), (u3, v3, ou3, ov3)):
        for j in range(SCH0):
            @pl.when(j < nch)
            def _(u=u, v=v, ou=ou, ov=ov, j=j):
                base = (cbase + j) * K
                pltpu.sync_copy(u.at[pl.ds(base, K)], uidx)
                pltpu.sync_copy(v.at[pl.ds(base, K)], vidx)
                pltpu.async_copy(z.at[uidx], zu, gsem)
                pltpu.async_copy(z.at[vidx], zv, gsem)
                pltpu.make_async_copy(z.at[uidx], zu, gsem).wait()
                pltpu.make_async_copy(z.at[vidx], zv, gsem).wait()
                pltpu.sync_copy(zu, ou.at[pl.ds(base, K)])
                pltpu.sync_copy(zv, ov.at[pl.ds(base, K)])


_score_call = functools.partial(
    pl.kernel,
    _score_body,
    out_type=tuple(jax.ShapeDtypeStruct((SPAD, OUT), jnp.float32)
                   for _ in range(8)),
    mesh=plsc.VectorSubcoreMesh(core_axis_name="c", subcore_axis_name="s",
                                num_cores=NC, num_subcores=NS),
    scratch_types=[
        pltpu.VMEM((K,), jnp.int32),
        pltpu.VMEM((K,), jnp.int32),
        pltpu.VMEM((K, OUT), jnp.float32),
        pltpu.VMEM((K, OUT), jnp.float32),
        pltpu.SemaphoreType.DMA,
    ],
)()


SROWB = 1024


def _sdot_body(zu0, zv0, zu1, zv1, zu2, zv2, zu3, zv3, s0, s1, s2, s3):
    s0[...] = jnp.sum(zu0[...] * zv0[...], axis=1, keepdims=True)
    s1[...] = jnp.sum(zu1[...] * zv1[...], axis=1, keepdims=True)
    s2[...] = jnp.sum(zu2[...] * zv2[...], axis=1, keepdims=True)
    s3[...] = jnp.sum(zu3[...] * zv3[...], axis=1, keepdims=True)


ROWB = 1000


def _dense1_body(agg, deg, w10, w11, b1s, w20, w21, p0, p1):
    a = agg[...]
    d = deg[...]
    dd0 = jnp.maximum(d[0, 0, :, 0:1] + d[0, 1, :, 0:1], 1.0)
    dd1 = jnp.maximum(d[1, 0, :, 0:1] + d[1, 1, :, 0:1], 1.0)
    n0 = (a[0, 0] + a[0, 1]) / dd0
    n1 = (a[1, 0] + a[1, 1]) / dd1
    h = (jnp.dot(n0, w10[...], preferred_element_type=jnp.float32)
         + jnp.dot(n1, w11[...], preferred_element_type=jnp.float32)
         + b1s[...])
    h = jnp.where(h >= 0, h, 0.01 * h)
    p0[...] = jnp.dot(h, w20[...], preferred_element_type=jnp.float32)
    p1[...] = jnp.dot(h, w21[...], preferred_element_type=jnp.float32)


def _dense2_body(agg, deg, b2s, z):
    a = agg[...]
    d = deg[...]
    dd0 = jnp.maximum(d[0, 0, :, 0:1] + d[0, 1, :, 0:1], 1.0)
    dd1 = jnp.maximum(d[1, 0, :, 0:1] + d[1, 1, :, 0:1], 1.0)
    z[...] = (a[0, 0] + a[0, 1]) / dd0 + (a[1, 0] + a[1, 1]) / dd1 + b2s[...]


def _pad_edges(e, tot, dummy_dst, flat=False):
    pad = tot - e.shape[1]
    src = jnp.concatenate([e[0], jnp.zeros((pad,), jnp.int32)])
    dst = jnp.concatenate([e[1], jnp.full((pad,), dummy_dst, jnp.int32)])
    if flat:
        return src, dst
    return src.reshape(-1, K), dst.reshape(-1, K)


def kernel(x, e0_rel0, e0_rel1, e1_rel0, e1_rel1,
           pos_rel0, pos_rel1, neg_rel0, neg_rel1,
           W1_rel0, W1_rel1, b1_rel0, b1_rel1,
           W2_rel0, W2_rel1, b2_rel0, b2_rel1):
    f32 = jnp.float32
    zer = jnp.zeros((RPT, IN), f32)
    zer8 = jnp.zeros((RPT,), f32)
    one8 = jnp.ones((K,), f32)

    s00, d00 = _pad_edges(e0_rel0, EPAD, N)
    s01, d01 = _pad_edges(e0_rel1, EPAD, N)
    s10, d10 = _pad_edges(e1_rel0, EPAD, N)
    s11, d11 = _pad_edges(e1_rel1, EPAD, N)

    agg1, deg1 = _seg_call(s00, d00, s01, d01, x, x, zer, zer8, one8)
    deg1 = deg1.reshape(2, NC, R, 1)

    b1s = (b1_rel0 + b1_rel1).reshape(1, HID)
    grid = (N // ROWB,)
    p0, p1 = pl.pallas_call(
        _dense1_body,
        grid=grid,
        in_specs=[
            pl.BlockSpec((2, NC, ROWB, IN), lambda i: (0, 0, i, 0)),
            pl.BlockSpec((2, NC, ROWB, 1), lambda i: (0, 0, i, 0)),
            pl.BlockSpec((IN, HID), lambda i: (0, 0)),
            pl.BlockSpec((IN, HID), lambda i: (0, 0)),
            pl.BlockSpec((1, HID), lambda i: (0, 0)),
            pl.BlockSpec((HID, OUT), lambda i: (0, 0)),
            pl.BlockSpec((HID, OUT), lambda i: (0, 0)),
        ],
        out_specs=[
            pl.BlockSpec((ROWB, OUT), lambda i: (i, 0)),
            pl.BlockSpec((ROWB, OUT), lambda i: (i, 0)),
        ],
        out_shape=[
            jax.ShapeDtypeStruct((N, OUT), f32),
            jax.ShapeDtypeStruct((N, OUT), f32),
        ],
    )(agg1, deg1, W1_rel0, W1_rel1, b1s, W2_rel0, W2_rel1)

    agg2, deg2 = _seg_call(s10, d10, s11, d11, p0, p1, zer, zer8, one8)
    deg2 = deg2.reshape(2, NC, R, 1)

    b2s = (b2_rel0 + b2_rel1).reshape(1, OUT)
    z = pl.pallas_call(
        _dense2_body,
        grid=grid,
        in_specs=[
            pl.BlockSpec((2, NC, ROWB, OUT), lambda i: (0, 0, i, 0)),
            pl.BlockSpec((2, NC, ROWB, 1), lambda i: (0, 0, i, 0)),
            pl.BlockSpec((1, OUT), lambda i: (0, 0)),
        ],
        out_specs=pl.BlockSpec((ROWB, OUT), lambda i: (i, 0)),
        out_shape=jax.ShapeDtypeStruct((N, OUT), f32),
    )(agg2, deg2, b2s)

    su0, tv0 = _pad_edges(pos_rel0, SPAD, 0, flat=True)
    su1, tv1 = _pad_edges(pos_rel1, SPAD, 0, flat=True)
    su2, tv2 = _pad_edges(neg_rel0, SPAD, 0, flat=True)
    su3, tv3 = _pad_edges(neg_rel1, SPAD, 0, flat=True)
    g = _score_call(z, su0, tv0, su1, tv1, su2, tv2, su3, tv3)
    sgrid = (SPAD // SROWB,)
    vec_spec = pl.BlockSpec((SROWB, OUT), lambda i: (i, 0))
    out_spec = pl.BlockSpec((SROWB, 1), lambda i: (i, 0))
    sc0, sc1, sc2, sc3 = pl.pallas_call(
        _sdot_body,
        grid=sgrid,
        in_specs=[vec_spec] * 8,
        out_specs=[out_spec] * 4,
        out_shape=[jax.ShapeDtypeStruct((SPAD, 1), jnp.float32)] * 4,
    )(*g)
    return (sc0[:P], sc1[:P], sc2[:P], sc3[:P])


# speedup vs baseline: 1.0839x; 1.0839x over previous
import functools

import jax
import jax.numpy as jnp
from jax import lax
from jax.experimental import pallas as pl
from jax.experimental.pallas import tpu as pltpu
from jax.experimental.pallas import tpu_sc as plsc

N = 10000
E = 160000
P = 20000
IN, HID, OUT = 128, 256, 128

NC, NS, L = 2, 16, 16
NW = NC * NS
K = 128
NBUF = 2

R = 10112
RD = 10240
ECH0, ECH1 = 64, 16
TCH = (ECH0 + ECH1) * NS
EPAD = TCH * K
SCH0, SCH1 = 7, 3
SPAD = (SCH0 + SCH1) * NS * K
RPT = R // NS
RPTD = RD // NS
DW = 1


def _seg_body(s0, d0, s1, d1, t0, t1, zer, zer8, one8,
              agg_out, deg_out,
              agg_sh, deg_sh, sidx, didx, r0, r1, o1v, gsem,
              ssem0, ssem1, dsem):
    c = lax.axis_index("c")
    s = lax.axis_index("s")
    rows = (r0, r1)
    ssems = (ssem0, ssem1)
    rbase = s * RPT
    cbase = jnp.where(c == 0, s * ECH0, NS * ECH0 + s * ECH1)
    nch = jnp.where(c == 0, ECH0, ECH1)
    ngrp = nch // NBUF

    pltpu.sync_copy(one8, o1v)
    pltpu.sync_copy(zer, agg_sh.at[pl.ds(rbase, RPT)])
    pltpu.sync_copy(zer8, deg_sh.at[pl.ds(s * RPTD, RPTD)])
    plsc.subcore_barrier()

    for rel in range(2):
        src = (s0, s1)[rel]
        dst = (d0, d1)[rel]
        tab = (t0, t1)[rel]

        @pl.when(c == 0)
        def _():
            pltpu.sync_copy(src.at[pl.ds(cbase, ECH0)], sidx)
            pltpu.sync_copy(dst.at[pl.ds(cbase, ECH0)], didx)

        @pl.when(c == 1)
        def _():
            pltpu.sync_copy(src.at[pl.ds(cbase, ECH1)], sidx.at[pl.ds(0, ECH1)])
            pltpu.sync_copy(dst.at[pl.ds(cbase, ECH1)], didx.at[pl.ds(0, ECH1)])

        pltpu.async_copy(tab.at[sidx.at[0]], r0, gsem)

        def chunk_group(p, carry, tab=tab):
            for b in range(NBUF):
                j = NBUF * p + b
                ob = 1 - b
                pltpu.make_async_copy(tab.at[sidx.at[j]], rows[b],
                                      gsem).wait()

                @pl.when(j >= 1)
                def _(ob=ob, j=j):
                    pltpu.make_async_copy(rows[ob],
                                          agg_sh.at[didx.at[j - 1]],
                                          ssems[ob]).wait()
                    pltpu.make_async_copy(o1v, deg_sh.at[didx.at[j - 1]],
                                          dsem).wait()

                pltpu.async_copy(rows[b], agg_sh.at[didx.at[j]], ssems[b],
                                 add=True)
                pltpu.async_copy(o1v, deg_sh.at[didx.at[j]], dsem, add=True)

                @pl.when(j + 1 < nch)
                def _(ob=ob, j=j, tab=tab):
                    pltpu.async_copy(tab.at[sidx.at[j + 1]], rows[ob], gsem)
            return carry

        lax.fori_loop(0, ngrp, chunk_group, 0)
        pltpu.make_async_copy(r1, agg_sh.at[didx.at[0]], ssem1).wait()
        pltpu.make_async_copy(o1v, deg_sh.at[didx.at[0]], dsem).wait()
        plsc.subcore_barrier()
        pltpu.sync_copy(agg_sh.at[pl.ds(rbase, RPT)],
                        agg_out.at[rel, c, pl.ds(rbase, RPT)])
        pltpu.sync_copy(deg_sh.at[pl.ds(s * RPTD, RPTD)],
                        deg_out.at[rel, c, pl.ds(s * RPTD, RPTD)])
        if rel == 0:
            pltpu.sync_copy(zer, agg_sh.at[pl.ds(rbase, RPT)])
            pltpu.sync_copy(zer8, deg_sh.at[pl.ds(s * RPTD, RPTD)])
            plsc.subcore_barrier()


_seg_call = functools.partial(
    pl.kernel,
    _seg_body,
    out_type=(
        jax.ShapeDtypeStruct((2, NC, R, IN), jnp.float32),
        jax.ShapeDtypeStruct((2, NC, RD), jnp.float32),
    ),
    mesh=plsc.VectorSubcoreMesh(core_axis_name="c", subcore_axis_name="s",
                                num_cores=NC, num_subcores=NS),
    scratch_types=[
        pltpu.VMEM_SHARED((R, IN), jnp.float32),
        pltpu.VMEM_SHARED((RD,), jnp.float32),
        pltpu.VMEM((ECH0, K), jnp.int32),
        pltpu.VMEM((ECH0, K), jnp.int32),
        pltpu.VMEM((K, IN), jnp.float32),
        pltpu.VMEM((K, IN), jnp.float32),
        pltpu.VMEM((K,), jnp.float32),
        pltpu.SemaphoreType.DMA,
        pltpu.SemaphoreType.DMA,
        pltpu.SemaphoreType.DMA,
        pltpu.SemaphoreType.DMA,
    ],
)()


def _score_body(z, u0, v0, u1, v1, u2, v2, u3, v3,
                ou0, ov0, ou1, ov1, ou2, ov2, ou3, ov3,
                uidx, vidx, zu, zv, gsem):
    c = lax.axis_index("c")
    s = lax.axis_index("s")
    cbase = jnp.where(c == 0, s * SCH0, NS * SCH0 + s * SCH1)
    nch = jnp.where(c == 0, SCH0, SCH1)

    for u, v, ou, ov in ((u0, v0, ou0, ov0), (u1, v1, ou1, ov1),
                         (u2, v2, ou2, ov2), (u3, v3, ou3, ov3)):
        for j in range(SCH0):
            @pl.when(j < nch)
            def _(u=u, v=v, ou=ou, ov=ov, j=j):
                base = (cbase + j) * K
                pltpu.sync_copy(u.at[pl.ds(base, K)], uidx)
                pltpu.sync_copy(v.at[pl.ds(base, K)], vidx)
                pltpu.async_copy(z.at[uidx], zu, gsem)
                pltpu.async_copy(z.at[vidx], zv, gsem)
                pltpu.make_async_copy(z.at[uidx], zu, gsem).wait()
                pltpu.make_async_copy(z.at[vidx], zv, gsem).wait()
                pltpu.sync_copy(zu, ou.at[pl.ds(base, K)])
                pltpu.sync_copy(zv, ov.at[pl.ds(base, K)])


_score_call = functools.partial(
    pl.kernel,
    _score_body,
    out_type=tuple(jax.ShapeDtypeStruct((SPAD, OUT), jnp.float32)
                   for _ in range(8)),
    mesh=plsc.VectorSubcoreMesh(core_axis_name="c", subcore_axis_name="s",
                                num_cores=NC, num_subcores=NS),
    scratch_types=[
        pltpu.VMEM((K,), jnp.int32),
        pltpu.VMEM((K,), jnp.int32),
        pltpu.VMEM((K, OUT), jnp.float32),
        pltpu.VMEM((K, OUT), jnp.float32),
        pltpu.SemaphoreType.DMA,
    ],
)()


SROWB = 1024


def _sdot_body(zu0, zv0, zu1, zv1, zu2, zv2, zu3, zv3, s0, s1, s2, s3):
    s0[...] = jnp.sum(zu0[...] * zv0[...], axis=1, keepdims=True)
    s1[...] = jnp.sum(zu1[...] * zv1[...], axis=1, keepdims=True)
    s2[...] = jnp.sum(zu2[...] * zv2[...], axis=1, keepdims=True)
    s3[...] = jnp.sum(zu3[...] * zv3[...], axis=1, keepdims=True)


ROWB = 1000


def _dense1_body(agg, deg, w10, w11, b1s, w20, w21, p0, p1):
    a = agg[...]
    d = deg[...]
    dd0 = jnp.maximum(d[0, 0, :, 0:1] + d[0, 1, :, 0:1], 1.0)
    dd1 = jnp.maximum(d[1, 0, :, 0:1] + d[1, 1, :, 0:1], 1.0)
    n0 = (a[0, 0] + a[0, 1]) / dd0
    n1 = (a[1, 0] + a[1, 1]) / dd1
    h = (jnp.dot(n0, w10[...], preferred_element_type=jnp.float32)
         + jnp.dot(n1, w11[...], preferred_element_type=jnp.float32)
         + b1s[...])
    h = jnp.where(h >= 0, h, 0.01 * h)
    p0[...] = jnp.dot(h, w20[...], preferred_element_type=jnp.float32)
    p1[...] = jnp.dot(h, w21[...], preferred_element_type=jnp.float32)


def _dense2_body(agg, deg, b2s, z):
    a = agg[...]
    d = deg[...]
    dd0 = jnp.maximum(d[0, 0, :, 0:1] + d[0, 1, :, 0:1], 1.0)
    dd1 = jnp.maximum(d[1, 0, :, 0:1] + d[1, 1, :, 0:1], 1.0)
    z[...] = (a[0, 0] + a[0, 1]) / dd0 + (a[1, 0] + a[1, 1]) / dd1 + b2s[...]


def _pad_edges(e, tot, dummy_dst, flat=False):
    pad = tot - e.shape[1]
    src = jnp.concatenate([e[0], jnp.zeros((pad,), jnp.int32)])
    dst = jnp.concatenate([e[1], jnp.full((pad,), dummy_dst, jnp.int32)])
    if flat:
        return src, dst
    return src.reshape(-1, K), dst.reshape(-1, K)


def kernel(x, e0_rel0, e0_rel1, e1_rel0, e1_rel1,
           pos_rel0, pos_rel1, neg_rel0, neg_rel1,
           W1_rel0, W1_rel1, b1_rel0, b1_rel1,
           W2_rel0, W2_rel1, b2_rel0, b2_rel1):
    f32 = jnp.float32
    zer = jnp.zeros((RPT, IN), f32)
    zer8 = jnp.zeros((RPTD,), f32)
    one8 = jnp.ones((K,), f32)

    s00, d00 = _pad_edges(e0_rel0, EPAD, N)
    s01, d01 = _pad_edges(e0_rel1, EPAD, N)
    s10, d10 = _pad_edges(e1_rel0, EPAD, N)
    s11, d11 = _pad_edges(e1_rel1, EPAD, N)

    agg1, deg1 = _seg_call(s00, d00, s01, d01, x, x, zer, zer8, one8)
    deg1 = deg1.reshape(2, NC, RD, 1)

    b1s = (b1_rel0 + b1_rel1).reshape(1, HID)
    grid = (N // ROWB,)
    p0, p1 = pl.pallas_call(
        _dense1_body,
        grid=grid,
        in_specs=[
            pl.BlockSpec((2, NC, ROWB, IN), lambda i: (0, 0, i, 0)),
            pl.BlockSpec((2, NC, ROWB, 1), lambda i: (0, 0, i, 0)),
            pl.BlockSpec((IN, HID), lambda i: (0, 0)),
            pl.BlockSpec((IN, HID), lambda i: (0, 0)),
            pl.BlockSpec((1, HID), lambda i: (0, 0)),
            pl.BlockSpec((HID, OUT), lambda i: (0, 0)),
            pl.BlockSpec((HID, OUT), lambda i: (0, 0)),
        ],
        out_specs=[
            pl.BlockSpec((ROWB, OUT), lambda i: (i, 0)),
            pl.BlockSpec((ROWB, OUT), lambda i: (i, 0)),
        ],
        out_shape=[
            jax.ShapeDtypeStruct((N, OUT), f32),
            jax.ShapeDtypeStruct((N, OUT), f32),
        ],
    )(agg1, deg1, W1_rel0, W1_rel1, b1s, W2_rel0, W2_rel1)

    agg2, deg2 = _seg_call(s10, d10, s11, d11, p0, p1, zer, zer8, one8)
    deg2 = deg2.reshape(2, NC, RD, 1)

    b2s = (b2_rel0 + b2_rel1).reshape(1, OUT)
    z = pl.pallas_call(
        _dense2_body,
        grid=grid,
        in_specs=[
            pl.BlockSpec((2, NC, ROWB, OUT), lambda i: (0, 0, i, 0)),
            pl.BlockSpec((2, NC, ROWB, 1), lambda i: (0, 0, i, 0)),
            pl.BlockSpec((1, OUT), lambda i: (0, 0)),
        ],
        out_specs=pl.BlockSpec((ROWB, OUT), lambda i: (i, 0)),
        out_shape=jax.ShapeDtypeStruct((N, OUT), f32),
    )(agg2, deg2, b2s)

    su0, tv0 = _pad_edges(pos_rel0, SPAD, 0, flat=True)
    su1, tv1 = _pad_edges(pos_rel1, SPAD, 0, flat=True)
    su2, tv2 = _pad_edges(neg_rel0, SPAD, 0, flat=True)
    su3, tv3 = _pad_edges(neg_rel1, SPAD, 0, flat=True)
    g = _score_call(z, su0, tv0, su1, tv1, su2, tv2, su3, tv3)
    sgrid = (SPAD // SROWB,)
    vec_spec = pl.BlockSpec((SROWB, OUT), lambda i: (i, 0))
    out_spec = pl.BlockSpec((SROWB, 1), lambda i: (i, 0))
    sc0, sc1, sc2, sc3 = pl.pallas_call(
        _sdot_body,
        grid=sgrid,
        in_specs=[vec_spec] * 8,
        out_specs=[out_spec] * 4,
        out_shape=[jax.ShapeDtypeStruct((SPAD, 1), jnp.float32)] * 4,
    )(*g)
    return (sc0[:P], sc1[:P], sc2[:P], sc3[:P])


# speedup vs baseline: 1.1027x; 1.0174x over previous
import functools

import jax
import jax.numpy as jnp
from jax import lax
from jax.experimental import pallas as pl
from jax.experimental.pallas import tpu as pltpu
from jax.experimental.pallas import tpu_sc as plsc

N = 10000
E = 160000
P = 20000
IN, HID, OUT = 128, 256, 128

NC, NS, L = 2, 16, 16
NW = NC * NS
K = 128
NBUF = 2

R = 10112
RD = 10240
ECH0, ECH1 = 64, 16
TCH = (ECH0 + ECH1) * NS
EPAD = TCH * K
SCH0, SCH1 = 7, 3
SPAD = (SCH0 + SCH1) * NS * K
RPT = R // NS
RPTD = RD // NS
DW = 1


def _seg_body(s0, d0, s1, d1, t0, t1, zer, zer8, one8,
              agg_out, deg_out,
              agg_sh, deg_sh, sidx, didx, r0, r1, o1v, gsem,
              ssem0, ssem1, dsem):
    c = lax.axis_index("c")
    s = lax.axis_index("s")
    rows = (r0, r1)
    ssems = (ssem0, ssem1)
    rbase = s * RPT
    cbase = jnp.where(c == 0, s * ECH0, NS * ECH0 + s * ECH1)
    nch = jnp.where(c == 0, ECH0, ECH1)
    ngrp = nch // NBUF

    pltpu.sync_copy(one8, o1v)
    pltpu.sync_copy(zer, agg_sh.at[pl.ds(rbase, RPT)])
    pltpu.sync_copy(zer8, deg_sh.at[pl.ds(s * RPTD, RPTD)])
    plsc.subcore_barrier()

    for rel in range(2):
        src = (s0, s1)[rel]
        dst = (d0, d1)[rel]
        tab = (t0, t1)[rel]

        @pl.when(c == 0)
        def _():
            pltpu.sync_copy(src.at[pl.ds(cbase, ECH0)], sidx)
            pltpu.sync_copy(dst.at[pl.ds(cbase, ECH0)], didx)

        @pl.when(c == 1)
        def _():
            pltpu.sync_copy(src.at[pl.ds(cbase, ECH1)], sidx.at[pl.ds(0, ECH1)])
            pltpu.sync_copy(dst.at[pl.ds(cbase, ECH1)], didx.at[pl.ds(0, ECH1)])

        pltpu.async_copy(tab.at[sidx.at[0]], r0, gsem)

        def chunk_group(p, carry, tab=tab):
            for b in range(NBUF):
                j = NBUF * p + b
                ob = 1 - b
                pltpu.make_async_copy(tab.at[sidx.at[j]], rows[b],
                                      gsem).wait()

                @pl.when(j >= 1)
                def _(ob=ob, j=j):
                    pltpu.make_async_copy(rows[ob],
                                          agg_sh.at[didx.at[j - 1]],
                                          ssems[ob]).wait()
                    pltpu.make_async_copy(o1v, deg_sh.at[didx.at[j - 1]],
                                          dsem).wait()

                pltpu.async_copy(rows[b], agg_sh.at[didx.at[j]], ssems[b],
                                 add=True)
                pltpu.async_copy(o1v, deg_sh.at[didx.at[j]], dsem, add=True)

                @pl.when(j + 1 < nch)
                def _(ob=ob, j=j, tab=tab):
                    pltpu.async_copy(tab.at[sidx.at[j + 1]], rows[ob], gsem)
            return carry

        lax.fori_loop(0, ngrp, chunk_group, 0)
        pltpu.make_async_copy(r1, agg_sh.at[didx.at[0]], ssem1).wait()
        pltpu.make_async_copy(o1v, deg_sh.at[didx.at[0]], dsem).wait()
        plsc.subcore_barrier()
        pltpu.sync_copy(agg_sh.at[pl.ds(rbase, RPT)],
                        agg_out.at[rel, c, pl.ds(rbase, RPT)])
        pltpu.sync_copy(deg_sh.at[pl.ds(s * RPTD, RPTD)],
                        deg_out.at[rel, c, pl.ds(s * RPTD, RPTD)])
        if rel == 0:
            pltpu.sync_copy(zer, agg_sh.at[pl.ds(rbase, RPT)])
            pltpu.sync_copy(zer8, deg_sh.at[pl.ds(s * RPTD, RPTD)])
            plsc.subcore_barrier()


_seg_call = functools.partial(
    pl.kernel,
    _seg_body,
    out_type=(
        jax.ShapeDtypeStruct((2, NC, R, IN), jnp.float32),
        jax.ShapeDtypeStruct((2, NC, RD), jnp.float32),
    ),
    mesh=plsc.VectorSubcoreMesh(core_axis_name="c", subcore_axis_name="s",
                                num_cores=NC, num_subcores=NS),
    scratch_types=[
        pltpu.VMEM_SHARED((R, IN), jnp.float32),
        pltpu.VMEM_SHARED((RD,), jnp.float32),
        pltpu.VMEM((ECH0, K), jnp.int32),
        pltpu.VMEM((ECH0, K), jnp.int32),
        pltpu.VMEM((K, IN), jnp.float32),
        pltpu.VMEM((K, IN), jnp.float32),
        pltpu.VMEM((K,), jnp.float32),
        pltpu.SemaphoreType.DMA,
        pltpu.SemaphoreType.DMA,
        pltpu.SemaphoreType.DMA,
        pltpu.SemaphoreType.DMA,
    ],
)()


def _score_body(z, u0, v0, u1, v1, u2, v2, u3, v3,
                ou0, ov0, ou1, ov1, ou2, ov2, ou3, ov3,
                ui0, vi0, ui1, vi1, zuA, zvA, zuB, zvB,
                g0sem, g1sem, w0sem, w1sem):
    c = lax.axis_index("c")
    s = lax.axis_index("s")
    cbase = jnp.where(c == 0, s * SCH0, NS * SCH0 + s * SCH1)
    nch = jnp.where(c == 0, SCH0, SCH1)
    uis = (ui0, ui1)
    vis = (vi0, vi1)
    zus = (zuA, zuB)
    zvs = (zvA, zvB)
    gsems = (g0sem, g1sem)
    wsems = (w0sem, w1sem)

    for u, v, ou, ov in ((u0, v0, ou0, ov0), (u1, v1, ou1, ov1),
                         (u2, v2, ou2, ov2), (u3, v3, ou3, ov3)):
        pltpu.sync_copy(u.at[pl.ds(cbase * K, K)], ui0)
        pltpu.sync_copy(v.at[pl.ds(cbase * K, K)], vi0)
        pltpu.async_copy(z.at[ui0], zuA, g0sem)
        pltpu.async_copy(z.at[vi0], zvA, g0sem)

        for j in range(SCH0):
            b = j % 2
            ob = 1 - b

            @pl.when(j < nch)
            def _(u=u, v=v, ou=ou, ov=ov, j=j, b=b, ob=ob):
                @pl.when(j + 1 < nch)
                def _(u=u, v=v, j=j, ob=ob):
                    nbase = (cbase + j + 1) * K
                    pltpu.sync_copy(u.at[pl.ds(nbase, K)], uis[ob])
                    pltpu.sync_copy(v.at[pl.ds(nbase, K)], vis[ob])

                @pl.when(j >= 1)
                def _(ou=ou, ov=ov, j=j, ob=ob):
                    pbase = (cbase + j - 1) * K
                    pltpu.make_async_copy(zus[ob], ou.at[pl.ds(pbase, K)],
                                          wsems[ob]).wait()
                    pltpu.make_async_copy(zvs[ob], ov.at[pl.ds(pbase, K)],
                                          wsems[ob]).wait()

                @pl.when(j + 1 < nch)
                def _(j=j, ob=ob):
                    pltpu.async_copy(z.at[uis[ob]], zus[ob], gsems[ob])
                    pltpu.async_copy(z.at[vis[ob]], zvs[ob], gsems[ob])

                pltpu.make_async_copy(z.at[uis[b]], zus[b], gsems[b]).wait()
                pltpu.make_async_copy(z.at[vis[b]], zvs[b], gsems[b]).wait()
                base = (cbase + j) * K
                pltpu.async_copy(zus[b], ou.at[pl.ds(base, K)], wsems[b])
                pltpu.async_copy(zvs[b], ov.at[pl.ds(base, K)], wsems[b])

        for par in range(2):
            @pl.when((nch - 1) % 2 == par)
            def _(ou=ou, ov=ov, par=par):
                lbase = (cbase + nch - 1) * K
                pltpu.make_async_copy(zus[par], ou.at[pl.ds(lbase, K)],
                                      wsems[par]).wait()
                pltpu.make_async_copy(zvs[par], ov.at[pl.ds(lbase, K)],
                                      wsems[par]).wait()


_score_call = functools.partial(
    pl.kernel,
    _score_body,
    out_type=tuple(jax.ShapeDtypeStruct((SPAD, OUT), jnp.float32)
                   for _ in range(8)),
    mesh=plsc.VectorSubcoreMesh(core_axis_name="c", subcore_axis_name="s",
                                num_cores=NC, num_subcores=NS),
    scratch_types=[
        pltpu.VMEM((K,), jnp.int32),
        pltpu.VMEM((K,), jnp.int32),
        pltpu.VMEM((K,), jnp.int32),
        pltpu.VMEM((K,), jnp.int32),
        pltpu.VMEM((K, OUT), jnp.float32),
        pltpu.VMEM((K, OUT), jnp.float32),
        pltpu.VMEM((K, OUT), jnp.float32),
        pltpu.VMEM((K, OUT), jnp.float32),
        pltpu.SemaphoreType.DMA,
        pltpu.SemaphoreType.DMA,
        pltpu.SemaphoreType.DMA,
        pltpu.SemaphoreType.DMA,
    ],
)()


SROWB = 1024


def _sdot_body(zu0, zv0, zu1, zv1, zu2, zv2, zu3, zv3, s0, s1, s2, s3):
    s0[...] = jnp.sum(zu0[...] * zv0[...], axis=1, keepdims=True)
    s1[...] = jnp.sum(zu1[...] * zv1[...], axis=1, keepdims=True)
    s2[...] = jnp.sum(zu2[...] * zv2[...], axis=1, keepdims=True)
    s3[...] = jnp.sum(zu3[...] * zv3[...], axis=1, keepdims=True)


ROWB = 1000


def _dense1_body(agg, deg, w10, w11, b1s, w20, w21, p0, p1):
    a = agg[...]
    d = deg[...]
    dd0 = jnp.maximum(d[0, 0, :, 0:1] + d[0, 1, :, 0:1], 1.0)
    dd1 = jnp.maximum(d[1, 0, :, 0:1] + d[1, 1, :, 0:1], 1.0)
    n0 = (a[0, 0] + a[0, 1]) / dd0
    n1 = (a[1, 0] + a[1, 1]) / dd1
    h = (jnp.dot(n0, w10[...], preferred_element_type=jnp.float32)
         + jnp.dot(n1, w11[...], preferred_element_type=jnp.float32)
         + b1s[...])
    h = jnp.where(h >= 0, h, 0.01 * h)
    p0[...] = jnp.dot(h, w20[...], preferred_element_type=jnp.float32)
    p1[...] = jnp.dot(h, w21[...], preferred_element_type=jnp.float32)


def _dense2_body(agg, deg, b2s, z):
    a = agg[...]
    d = deg[...]
    dd0 = jnp.maximum(d[0, 0, :, 0:1] + d[0, 1, :, 0:1], 1.0)
    dd1 = jnp.maximum(d[1, 0, :, 0:1] + d[1, 1, :, 0:1], 1.0)
    z[...] = (a[0, 0] + a[0, 1]) / dd0 + (a[1, 0] + a[1, 1]) / dd1 + b2s[...]


def _pad_edges(e, tot, dummy_dst, flat=False):
    pad = tot - e.shape[1]
    src = jnp.concatenate([e[0], jnp.zeros((pad,), jnp.int32)])
    dst = jnp.concatenate([e[1], jnp.full((pad,), dummy_dst, jnp.int32)])
    if flat:
        return src, dst
    return src.reshape(-1, K), dst.reshape(-1, K)


def kernel(x, e0_rel0, e0_rel1, e1_rel0, e1_rel1,
           pos_rel0, pos_rel1, neg_rel0, neg_rel1,
           W1_rel0, W1_rel1, b1_rel0, b1_rel1,
           W2_rel0, W2_rel1, b2_rel0, b2_rel1):
    f32 = jnp.float32
    zer = jnp.zeros((RPT, IN), f32)
    zer8 = jnp.zeros((RPTD,), f32)
    one8 = jnp.ones((K,), f32)

    s00, d00 = _pad_edges(e0_rel0, EPAD, N)
    s01, d01 = _pad_edges(e0_rel1, EPAD, N)
    s10, d10 = _pad_edges(e1_rel0, EPAD, N)
    s11, d11 = _pad_edges(e1_rel1, EPAD, N)

    agg1, deg1 = _seg_call(s00, d00, s01, d01, x, x, zer, zer8, one8)
    deg1 = deg1.reshape(2, NC, RD, 1)

    b1s = (b1_rel0 + b1_rel1).reshape(1, HID)
    grid = (N // ROWB,)
    p0, p1 = pl.pallas_call(
        _dense1_body,
        grid=grid,
        in_specs=[
            pl.BlockSpec((2, NC, ROWB, IN), lambda i: (0, 0, i, 0)),
            pl.BlockSpec((2, NC, ROWB, 1), lambda i: (0, 0, i, 0)),
            pl.BlockSpec((IN, HID), lambda i: (0, 0)),
            pl.BlockSpec((IN, HID), lambda i: (0, 0)),
            pl.BlockSpec((1, HID), lambda i: (0, 0)),
            pl.BlockSpec((HID, OUT), lambda i: (0, 0)),
            pl.BlockSpec((HID, OUT), lambda i: (0, 0)),
        ],
        out_specs=[
            pl.BlockSpec((ROWB, OUT), lambda i: (i, 0)),
            pl.BlockSpec((ROWB, OUT), lambda i: (i, 0)),
        ],
        out_shape=[
            jax.ShapeDtypeStruct((N, OUT), f32),
            jax.ShapeDtypeStruct((N, OUT), f32),
        ],
    )(agg1, deg1, W1_rel0, W1_rel1, b1s, W2_rel0, W2_rel1)

    agg2, deg2 = _seg_call(s10, d10, s11, d11, p0, p1, zer, zer8, one8)
    deg2 = deg2.reshape(2, NC, RD, 1)

    b2s = (b2_rel0 + b2_rel1).reshape(1, OUT)
    z = pl.pallas_call(
        _dense2_body,
        grid=grid,
        in_specs=[
            pl.BlockSpec((2, NC, ROWB, OUT), lambda i: (0, 0, i, 0)),
            pl.BlockSpec((2, NC, ROWB, 1), lambda i: (0, 0, i, 0)),
            pl.BlockSpec((1, OUT), lambda i: (0, 0)),
        ],
        out_specs=pl.BlockSpec((ROWB, OUT), lambda i: (i, 0)),
        out_shape=jax.ShapeDtypeStruct((N, OUT), f32),
    )(agg2, deg2, b2s)

    su0, tv0 = _pad_edges(pos_rel0, SPAD, 0, flat=True)
    su1, tv1 = _pad_edges(pos_rel1, SPAD, 0, flat=True)
    su2, tv2 = _pad_edges(neg_rel0, SPAD, 0, flat=True)
    su3, tv3 = _pad_edges(neg_rel1, SPAD, 0, flat=True)
    g = _score_call(z, su0, tv0, su1, tv1, su2, tv2, su3, tv3)
    sgrid = (SPAD // SROWB,)
    vec_spec = pl.BlockSpec((SROWB, OUT), lambda i: (i, 0))
    out_spec = pl.BlockSpec((SROWB, 1), lambda i: (i, 0))
    sc0, sc1, sc2, sc3 = pl.pallas_call(
        _sdot_body,
        grid=sgrid,
        in_specs=[vec_spec] * 8,
        out_specs=[out_spec] * 4,
        out_shape=[jax.ShapeDtypeStruct((SPAD, 1), jnp.float32)] * 4,
    )(*g)
    return (sc0[:P], sc1[:P], sc2[:P], sc3[:P])
